# baseline (speedup 1.0000x reference)
import functools

import jax
import jax.numpy as jnp
import numpy as np
from jax import lax
from jax.experimental import pallas as pl
from jax.experimental.pallas import tpu as pltpu
from jax.experimental.pallas import tpu_sc as plsc

_NC, _NS = 2, 16
_NW = _NC * _NS
_D = 32
_DPAD = 128
_V = 1000000
_NBLK = (_V + _DPAD - 1) // _DPAD
_GROWS = _NBLK * (_DPAD // 4)
_NB = 16384
_NCOL = 26
_BPW = _NB // _NW
_CHUNK = 128
_JPW = _BPW // _CHUNK
_NSTEP = _NCOL * _JPW

_mesh = plsc.VectorSubcoreMesh(core_axis_name="c", subcore_axis_name="s")


def _iota():
    return jnp.arange(16, dtype=jnp.int32)


@functools.partial(
    pl.kernel,
    mesh=_mesh,
    out_type=jax.ShapeDtypeStruct((_GROWS, _DPAD), jnp.float32),
    scratch_types=[
        pltpu.VMEM((2, _D, _DPAD), jnp.float32),
        pltpu.VMEM((2, _D, _DPAD), jnp.float32),
        pltpu.SemaphoreType.DMA,
        pltpu.SemaphoreType.DMA,
    ],
    compiler_params=pltpu.CompilerParams(
        use_tc_tiling_on_sc=True, disable_bounds_checks=True,
        needs_layout_passes=False),
)
def _format_table(tabt_hbm, g_hbm, s_v, t_v, isem, osem):
    wid = lax.axis_index("s") * _NC + lax.axis_index("c")
    nmine = lax.select(wid < _NBLK - (_NBLK // _NW) * _NW,
                       _NBLK // _NW + 1, _NBLK // _NW)

    def fire_in(m):
        blk = wid + m * _NW
        pltpu.async_copy(
            tabt_hbm.at[:, pl.ds(blk * _DPAD, _DPAD)],
            s_v.at[lax.rem(m, 2)],
            isem)

    @pl.when(nmine > 0)
    def _():
        fire_in(0)

    def block_step(m, carry):
        buf = lax.rem(m, 2)
        blk = wid + m * _NW
        pltpu.make_async_copy(
            tabt_hbm.at[:, pl.ds(0, _DPAD)], s_v.at[0], isem).wait()

        @pl.when(m + 1 < nmine)
        def _():
            fire_in(m + 1)

        @pl.when(m >= 2)
        def _():
            pltpu.make_async_copy(
                t_v.at[0], g_hbm.at[pl.ds(0, _D), :], osem).wait()

        rowpat = _iota() >> 2
        colpat = (_iota() & 3) * 32
        for d in range(_D):
            for g in range(_DPAD // 16):
                v = s_v[buf, d, pl.ds(16 * g, 16)]
                plsc.store_scatter(
                    t_v.at[buf],
                    [rowpat + (4 * g), colpat + d],
                    v)

        pltpu.async_copy(
            t_v.at[buf],
            g_hbm.at[pl.ds(blk * _D, _D), :],
            osem)
        return carry

    lax.fori_loop(0, nmine, block_step, 0)
    @pl.when(nmine >= 1)
    def _():
        pltpu.make_async_copy(
            t_v.at[0], g_hbm.at[pl.ds(0, _D), :], osem).wait()

    @pl.when(nmine >= 2)
    def _():
        pltpu.make_async_copy(
            t_v.at[0], g_hbm.at[pl.ds(0, _D), :], osem).wait()


@functools.partial(
    pl.kernel,
    mesh=_mesh,
    out_type=jax.ShapeDtypeStruct((_NCOL, _D, _NB), jnp.float32),
    scratch_types=[
        pltpu.VMEM((_NCOL, _BPW), jnp.int32),
        pltpu.VMEM((_NCOL, _BPW), jnp.int32),
        pltpu.VMEM((2, _CHUNK, _DPAD), jnp.float32),
        pltpu.VMEM((2, _D, _CHUNK), jnp.float32),
        pltpu.SemaphoreType.DMA,
        pltpu.SemaphoreType.DMA,
    ],
    compiler_params=pltpu.CompilerParams(
        use_tc_tiling_on_sc=True, needs_layout_passes=False),
)
def _embed_gather(idxt_hbm, g_hbm, out_hbm, idx_v, wid_v, g_v, a_v, gsem, osem):
    wid = lax.axis_index("s") * _NC + lax.axis_index("c")
    b0 = wid * _BPW
    pltpu.sync_copy(idxt_hbm.at[:, pl.ds(b0, _BPW)], idx_v)

    def shift_row(c, carry):
        def shift_grp(k, carry2):
            v = idx_v[c, pl.ds(16 * k, 16)]
            wid_v[c, pl.ds(16 * k, 16)] = v >> 2
            return carry2
        lax.fori_loop(0, _BPW // 16, shift_grp, 0)
        return carry
    lax.fori_loop(0, _NCOL, shift_row, 0)

    def fire_gather(t):
        c = lax.div(t, _JPW)
        j = lax.rem(t, _JPW)
        pltpu.async_copy(
            g_hbm.at[wid_v.at[c, pl.ds(j * _CHUNK, _CHUNK)]],
            g_v.at[lax.rem(t, 2)],
            gsem)

    fire_gather(0)

    def step(t, carry):
        buf = lax.rem(t, 2)
        pltpu.make_async_copy(
            g_hbm.at[wid_v.at[0, pl.ds(0, _CHUNK)]], g_v.at[0], gsem).wait()

        @pl.when(t + 1 < _NSTEP)
        def _():
            fire_gather(t + 1)

        @pl.when(t >= 2)
        def _():
            pltpu.make_async_copy(
                a_v.at[0], out_hbm.at[0, :, pl.ds(b0, _CHUNK)], osem).wait()

        c = lax.div(t, _JPW)
        j = lax.rem(t, _JPW)

        for k in range(_CHUNK // 16):
            idxv = idx_v[c, pl.ds(j * _CHUNK + 16 * k, 16)]
            lane0 = (idxv & 3) * 32
            rows = _iota() + (16 * k)
            for d in range(_D):
                vals = plsc.load_gather(g_v.at[buf], [rows, lane0 + d])
                a_v[buf, d, pl.ds(16 * k, 16)] = vals

        pltpu.async_copy(
            a_v.at[buf],
            out_hbm.at[c, :, pl.ds(b0 + j * _CHUNK, _CHUNK)],
            osem)
        return carry

    lax.fori_loop(0, _NSTEP, step, 0)
    pltpu.make_async_copy(
        a_v.at[0], out_hbm.at[0, :, pl.ds(b0, _CHUNK)], osem).wait()
    pltpu.make_async_copy(
        a_v.at[0], out_hbm.at[0, :, pl.ds(b0, _CHUNK)], osem).wait()


def kernel(input_idx, embedding_table):
    g = _format_table(embedding_table.T)
    out_t = _embed_gather(input_idx.T, g)
    return out_t.transpose(2, 0, 1)


# speedup vs baseline: 1.0000x; 1.0000x over previous
import functools

import jax
import jax.numpy as jnp
import numpy as np
from jax import lax
from jax.experimental import pallas as pl
from jax.experimental.pallas import tpu as pltpu
from jax.experimental.pallas import tpu_sc as plsc

_NC, _NS = 2, 16
_NW = _NC * _NS
_D = 32
_DPAD = 128
_V = 1000000
_NBLK = (_V + _DPAD - 1) // _DPAD
_GROWS = _NBLK * (_DPAD // 4)
_NB = 16384
_NCOL = 26
_BPW = _NB // _NW
_CHUNK = 128
_JPW = _BPW // _CHUNK
_NSTEP = _NCOL * _JPW

_mesh = plsc.VectorSubcoreMesh(core_axis_name="c", subcore_axis_name="s")


def _iota():
    return jnp.arange(16, dtype=jnp.int32)


@functools.partial(
    pl.kernel,
    mesh=_mesh,
    out_type=jax.ShapeDtypeStruct((_GROWS, _DPAD), jnp.float32),
    scratch_types=[
        pltpu.VMEM((2, _D, _DPAD), jnp.float32),
        pltpu.VMEM((2, _D, _DPAD), jnp.float32),
        pltpu.SemaphoreType.DMA,
        pltpu.SemaphoreType.DMA,
    ],
    compiler_params=pltpu.CompilerParams(
        use_tc_tiling_on_sc=True, disable_bounds_checks=True,
        needs_layout_passes=False),
)
def _format_table(tabt_hbm, g_hbm, s_v, t_v, isem, osem):
    wid = lax.axis_index("s") * _NC + lax.axis_index("c")
    nmine = lax.select(wid < _NBLK - (_NBLK // _NW) * _NW,
                       _NBLK // _NW + 1, _NBLK // _NW)

    def fire_in(m):
        blk = wid + m * _NW
        pltpu.async_copy(
            tabt_hbm.at[:, pl.ds(blk * _DPAD, _DPAD)],
            s_v.at[lax.rem(m, 2)],
            isem)

    @pl.when(nmine > 0)
    def _():
        fire_in(0)

    rowpat = _iota() >> 2
    colpat = (_iota() & 3) * 32
    rowpats = [rowpat + (4 * g) for g in range(_DPAD // 16)]
    colpats = [colpat + d for d in range(_D)]

    def block_step(m, carry):
        buf = lax.rem(m, 2)
        blk = wid + m * _NW
        pltpu.make_async_copy(
            tabt_hbm.at[:, pl.ds(0, _DPAD)], s_v.at[0], isem).wait()

        @pl.when(m + 1 < nmine)
        def _():
            fire_in(m + 1)

        @pl.when(m >= 2)
        def _():
            pltpu.make_async_copy(
                t_v.at[0], g_hbm.at[pl.ds(0, _D), :], osem).wait()

        for d in range(_D):
            for g in range(_DPAD // 16):
                v = s_v[buf, d, pl.ds(16 * g, 16)]
                plsc.store_scatter(t_v.at[buf], [rowpats[g], colpats[d]], v)

        pltpu.async_copy(
            t_v.at[buf],
            g_hbm.at[pl.ds(blk * _D, _D), :],
            osem)
        return carry

    lax.fori_loop(0, nmine, block_step, 0)
    @pl.when(nmine >= 1)
    def _():
        pltpu.make_async_copy(
            t_v.at[0], g_hbm.at[pl.ds(0, _D), :], osem).wait()

    @pl.when(nmine >= 2)
    def _():
        pltpu.make_async_copy(
            t_v.at[0], g_hbm.at[pl.ds(0, _D), :], osem).wait()


@functools.partial(
    pl.kernel,
    mesh=_mesh,
    out_type=jax.ShapeDtypeStruct((_NCOL, _D, _NB), jnp.float32),
    scratch_types=[
        pltpu.VMEM((_NCOL, _BPW), jnp.int32),
        pltpu.VMEM((_NCOL, _BPW), jnp.int32),
        pltpu.VMEM((4, _CHUNK, _DPAD), jnp.float32),
        pltpu.VMEM((2, _D, _CHUNK), jnp.float32),
        pltpu.SemaphoreType.DMA,
        pltpu.SemaphoreType.DMA,
    ],
    compiler_params=pltpu.CompilerParams(
        use_tc_tiling_on_sc=True, needs_layout_passes=False),
)
def _embed_gather(idxt_hbm, g_hbm, out_hbm, idx_v, wid_v, g_v, a_v, gsem, osem):
    wid = lax.axis_index("s") * _NC + lax.axis_index("c")
    b0 = wid * _BPW
    pltpu.sync_copy(idxt_hbm.at[:, pl.ds(b0, _BPW)], idx_v)

    def shift_row(c, carry):
        def shift_grp(k, carry2):
            v = idx_v[c, pl.ds(16 * k, 16)]
            wid_v[c, pl.ds(16 * k, 16)] = v >> 2
            return carry2
        lax.fori_loop(0, _BPW // 16, shift_grp, 0)
        return carry
    lax.fori_loop(0, _NCOL, shift_row, 0)

    def fire_gather(t):
        c = lax.div(t, _JPW)
        j = lax.rem(t, _JPW)
        pltpu.async_copy(
            g_hbm.at[wid_v.at[c, pl.ds(j * _CHUNK, _CHUNK)]],
            g_v.at[lax.rem(t, 4)],
            gsem)

    fire_gather(0)
    fire_gather(1)
    fire_gather(2)

    rowvecs = [_iota() + (16 * k) for k in range(_CHUNK // 16)]

    def step(t, carry):
        buf = lax.rem(t, 4)
        pltpu.make_async_copy(
            g_hbm.at[wid_v.at[0, pl.ds(0, _CHUNK)]], g_v.at[0], gsem).wait()

        @pl.when(t + 3 < _NSTEP)
        def _():
            fire_gather(t + 3)

        @pl.when(t >= 2)
        def _():
            pltpu.make_async_copy(
                a_v.at[0], out_hbm.at[0, :, pl.ds(b0, _CHUNK)], osem).wait()

        c = lax.div(t, _JPW)
        j = lax.rem(t, _JPW)

        buf2 = lax.rem(t, 2)
        for k in range(_CHUNK // 16):
            idxv = idx_v[c, pl.ds(j * _CHUNK + 16 * k, 16)]
            lane0 = (idxv & 3) * 32
            rows = rowvecs[k]
            for d in range(_D):
                vals = plsc.load_gather(g_v.at[buf], [rows, lane0 + d])
                a_v[buf2, d, pl.ds(16 * k, 16)] = vals

        pltpu.async_copy(
            a_v.at[buf2],
            out_hbm.at[c, :, pl.ds(b0 + j * _CHUNK, _CHUNK)],
            osem)
        return carry

    lax.fori_loop(0, _NSTEP, step, 0)
    pltpu.make_async_copy(
        a_v.at[0], out_hbm.at[0, :, pl.ds(b0, _CHUNK)], osem).wait()
    pltpu.make_async_copy(
        a_v.at[0], out_hbm.at[0, :, pl.ds(b0, _CHUNK)], osem).wait()


def kernel(input_idx, embedding_table):
    g = _format_table(embedding_table.T)
    out_t = _embed_gather(input_idx.T, g)
    return out_t.transpose(2, 0, 1)


# speedup vs baseline: 1.1684x; 1.1683x over previous
import functools

import jax
import jax.numpy as jnp
import numpy as np
from jax import lax
from jax.experimental import pallas as pl
from jax.experimental.pallas import tpu as pltpu
from jax.experimental.pallas import tpu_sc as plsc

_NC, _NS = 2, 16
_NW = _NC * _NS
_D = 32
_DPAD = 128
_V = 1000000
_NBLK = (_V + _DPAD - 1) // _DPAD
_GROWS = _NBLK * (_DPAD // 4)
_NB = 16384
_NCOL = 26
_BPW = _NB // _NW
_CHUNK = 128
_JPW = _BPW // _CHUNK
_NSTEP = _NCOL * _JPW

_mesh = plsc.VectorSubcoreMesh(core_axis_name="c", subcore_axis_name="s")


def _iota():
    return jnp.arange(16, dtype=jnp.int32)


@functools.partial(
    pl.kernel,
    mesh=_mesh,
    out_type=jax.ShapeDtypeStruct((_GROWS, _DPAD), jnp.float32),
    scratch_types=[
        pltpu.VMEM((2, _D, _DPAD), jnp.float32),
        pltpu.VMEM((2, _D, _DPAD), jnp.float32),
        pltpu.SemaphoreType.DMA,
        pltpu.SemaphoreType.DMA,
    ],
    compiler_params=pltpu.CompilerParams(
        use_tc_tiling_on_sc=True, disable_bounds_checks=True,
        needs_layout_passes=False),
)
def _format_table(tabt_hbm, g_hbm, s_v, t_v, isem, osem):
    wid = lax.axis_index("s") * _NC + lax.axis_index("c")
    nmine = lax.select(wid < _NBLK - (_NBLK // _NW) * _NW,
                       _NBLK // _NW + 1, _NBLK // _NW)

    def fire_in(m):
        blk = wid + m * _NW
        pltpu.async_copy(
            tabt_hbm.at[:, pl.ds(blk * _DPAD, _DPAD)],
            s_v.at[lax.rem(m, 2)],
            isem)

    @pl.when(nmine > 0)
    def _():
        fire_in(0)

    rowpat = _iota() >> 2
    colpat = (_iota() & 3) * 32
    rowpats = [rowpat + (4 * g) for g in range(_DPAD // 16)]
    colpats = [colpat + d for d in range(_D)]

    def block_step(m, carry):
        buf = lax.rem(m, 2)
        blk = wid + m * _NW
        pltpu.make_async_copy(
            tabt_hbm.at[:, pl.ds(0, _DPAD)], s_v.at[0], isem).wait()

        @pl.when(m + 1 < nmine)
        def _():
            fire_in(m + 1)

        @pl.when(m >= 2)
        def _():
            pltpu.make_async_copy(
                t_v.at[0], g_hbm.at[pl.ds(0, _D), :], osem).wait()

        for d in range(_D):
            vs = [s_v[buf, d, pl.ds(16 * g, 16)] for g in range(_DPAD // 16)]
            for g in range(_DPAD // 16):
                plsc.store_scatter(t_v.at[buf], [rowpats[g], colpats[d]], vs[g])

        pltpu.async_copy(
            t_v.at[buf],
            g_hbm.at[pl.ds(blk * _D, _D), :],
            osem)
        return carry

    lax.fori_loop(0, nmine, block_step, 0)
    @pl.when(nmine >= 1)
    def _():
        pltpu.make_async_copy(
            t_v.at[0], g_hbm.at[pl.ds(0, _D), :], osem).wait()

    @pl.when(nmine >= 2)
    def _():
        pltpu.make_async_copy(
            t_v.at[0], g_hbm.at[pl.ds(0, _D), :], osem).wait()


@functools.partial(
    pl.kernel,
    mesh=_mesh,
    out_type=jax.ShapeDtypeStruct((_NCOL, _D, _NB), jnp.float32),
    scratch_types=[
        pltpu.VMEM((_NCOL, _BPW), jnp.int32),
        pltpu.VMEM((_NCOL, _BPW), jnp.int32),
        pltpu.VMEM((4, _CHUNK, _DPAD), jnp.float32),
        pltpu.VMEM((2, _D, _CHUNK), jnp.float32),
        pltpu.SemaphoreType.DMA,
        pltpu.SemaphoreType.DMA,
    ],
    compiler_params=pltpu.CompilerParams(
        use_tc_tiling_on_sc=True, needs_layout_passes=False),
)
def _embed_gather(idxt_hbm, g_hbm, out_hbm, idx_v, wid_v, g_v, a_v, gsem, osem):
    wid = lax.axis_index("s") * _NC + lax.axis_index("c")
    b0 = wid * _BPW
    pltpu.sync_copy(idxt_hbm.at[:, pl.ds(b0, _BPW)], idx_v)

    def shift_row(c, carry):
        def shift_grp(k, carry2):
            v = idx_v[c, pl.ds(16 * k, 16)]
            wid_v[c, pl.ds(16 * k, 16)] = v >> 2
            return carry2
        lax.fori_loop(0, _BPW // 16, shift_grp, 0)
        return carry
    lax.fori_loop(0, _NCOL, shift_row, 0)

    def fire_gather(t):
        c = lax.div(t, _JPW)
        j = lax.rem(t, _JPW)
        pltpu.async_copy(
            g_hbm.at[wid_v.at[c, pl.ds(j * _CHUNK, _CHUNK)]],
            g_v.at[lax.rem(t, 4)],
            gsem)

    fire_gather(0)
    fire_gather(1)
    fire_gather(2)

    rowvecs = [_iota() + (16 * k) for k in range(_CHUNK // 16)]

    def step(t, carry):
        buf = lax.rem(t, 4)
        pltpu.make_async_copy(
            g_hbm.at[wid_v.at[0, pl.ds(0, _CHUNK)]], g_v.at[0], gsem).wait()

        @pl.when(t + 3 < _NSTEP)
        def _():
            fire_gather(t + 3)

        @pl.when(t >= 2)
        def _():
            pltpu.make_async_copy(
                a_v.at[0], out_hbm.at[0, :, pl.ds(b0, _CHUNK)], osem).wait()

        c = lax.div(t, _JPW)
        j = lax.rem(t, _JPW)

        buf2 = lax.rem(t, 2)
        for k in range(_CHUNK // 16):
            idxv = idx_v[c, pl.ds(j * _CHUNK + 16 * k, 16)]
            lane0 = (idxv & 3) * 32
            rows = rowvecs[k]
            for d0 in range(0, _D, 8):
                vals = [plsc.load_gather(g_v.at[buf], [rows, lane0 + (d0 + u)])
                        for u in range(8)]
                for u in range(8):
                    a_v[buf2, d0 + u, pl.ds(16 * k, 16)] = vals[u]

        pltpu.async_copy(
            a_v.at[buf2],
            out_hbm.at[c, :, pl.ds(b0 + j * _CHUNK, _CHUNK)],
            osem)
        return carry

    lax.fori_loop(0, _NSTEP, step, 0)
    pltpu.make_async_copy(
        a_v.at[0], out_hbm.at[0, :, pl.ds(b0, _CHUNK)], osem).wait()
    pltpu.make_async_copy(
        a_v.at[0], out_hbm.at[0, :, pl.ds(b0, _CHUNK)], osem).wait()


def kernel(input_idx, embedding_table):
    g = _format_table(embedding_table.T)
    out_t = _embed_gather(input_idx.T, g)
    return out_t.transpose(2, 0, 1)


# speedup vs baseline: 1.1684x; 1.0000x over previous
import functools

import jax
import jax.numpy as jnp
import numpy as np
from jax import lax
from jax.experimental import pallas as pl
from jax.experimental.pallas import tpu as pltpu
from jax.experimental.pallas import tpu_sc as plsc

_NC, _NS = 2, 16
_NW = _NC * _NS
_D = 32
_DPAD = 128
_V = 1000000
_NBLK = (_V + _DPAD - 1) // _DPAD
_GROWS = _NBLK * (_DPAD // 4)
_NB = 16384
_NCOL = 26
_BPW = _NB // _NW
_CHUNK = 128
_JPW = _BPW // _CHUNK
_NSTEP = _NCOL * _JPW

_mesh = plsc.VectorSubcoreMesh(core_axis_name="c", subcore_axis_name="s")


def _iota():
    return jnp.arange(16, dtype=jnp.int32)


@functools.partial(
    pl.kernel,
    mesh=_mesh,
    out_type=jax.ShapeDtypeStruct((_GROWS, _DPAD), jnp.float32),
    scratch_types=[
        pltpu.VMEM((4, _D, _DPAD), jnp.float32),
        pltpu.VMEM((2, _D, _DPAD), jnp.float32),
        pltpu.SemaphoreType.DMA,
        pltpu.SemaphoreType.DMA,
    ],
    compiler_params=pltpu.CompilerParams(
        use_tc_tiling_on_sc=True, disable_bounds_checks=True,
        needs_layout_passes=False),
)
def _format_table(tabt_hbm, g_hbm, s_v, t_v, isem, osem):
    wid = lax.axis_index("s") * _NC + lax.axis_index("c")
    nmine = lax.select(wid < _NBLK - (_NBLK // _NW) * _NW,
                       _NBLK // _NW + 1, _NBLK // _NW)

    def fire_in(m):
        blk = wid + m * _NW
        pltpu.async_copy(
            tabt_hbm.at[:, pl.ds(blk * _DPAD, _DPAD)],
            s_v.at[lax.rem(m, 4)],
            isem)

    for mm in range(3):
        @pl.when(mm < nmine)
        def _(mm=mm):
            fire_in(mm)

    rowpat = _iota() >> 2
    colpat = (_iota() & 3) * 32
    rowpats = [rowpat + (4 * g) for g in range(_DPAD // 16)]
    colpats = [colpat + d for d in range(_D)]

    def block_step(m, carry):
        buf = lax.rem(m, 4)
        blk = wid + m * _NW
        pltpu.make_async_copy(
            tabt_hbm.at[:, pl.ds(0, _DPAD)], s_v.at[0, :, pl.ds(0, _DPAD)],
            isem).wait()

        @pl.when(m + 3 < nmine)
        def _():
            fire_in(m + 3)

        @pl.when(m >= 2)
        def _():
            pltpu.make_async_copy(
                t_v.at[0], g_hbm.at[pl.ds(0, _D), :], osem).wait()

        for d in range(_D):
            vs = [s_v[buf, d, pl.ds(16 * g, 16)] for g in range(_DPAD // 16)]
            for g in range(_DPAD // 16):
                plsc.store_scatter(t_v.at[buf], [rowpats[g], colpats[d]], vs[g])

        pltpu.async_copy(
            t_v.at[buf],
            g_hbm.at[pl.ds(blk * _D, _D), :],
            osem)
        return carry

    lax.fori_loop(0, nmine, block_step, 0)
    @pl.when(nmine >= 1)
    def _():
        pltpu.make_async_copy(
            t_v.at[0], g_hbm.at[pl.ds(0, _D), :], osem).wait()

    @pl.when(nmine >= 2)
    def _():
        pltpu.make_async_copy(
            t_v.at[0], g_hbm.at[pl.ds(0, _D), :], osem).wait()


@functools.partial(
    pl.kernel,
    mesh=_mesh,
    out_type=jax.ShapeDtypeStruct((_NCOL, _D, _NB), jnp.float32),
    scratch_types=[
        pltpu.VMEM((_NCOL, _BPW), jnp.int32),
        pltpu.VMEM((_NCOL, _BPW), jnp.int32),
        pltpu.VMEM((4, _CHUNK, _DPAD), jnp.float32),
        pltpu.VMEM((2, _D, _CHUNK), jnp.float32),
        pltpu.SemaphoreType.DMA,
        pltpu.SemaphoreType.DMA,
    ],
    compiler_params=pltpu.CompilerParams(
        use_tc_tiling_on_sc=True, needs_layout_passes=False),
)
def _embed_gather(idxt_hbm, g_hbm, out_hbm, idx_v, wid_v, g_v, a_v, gsem, osem):
    wid = lax.axis_index("s") * _NC + lax.axis_index("c")
    b0 = wid * _BPW
    pltpu.sync_copy(idxt_hbm.at[:, pl.ds(b0, _BPW)], idx_v)

    def shift_row(c, carry):
        def shift_grp(k, carry2):
            v = idx_v[c, pl.ds(16 * k, 16)]
            wid_v[c, pl.ds(16 * k, 16)] = v >> 2
            return carry2
        lax.fori_loop(0, _BPW // 16, shift_grp, 0)
        return carry
    lax.fori_loop(0, _NCOL, shift_row, 0)

    def fire_gather(t):
        c = lax.div(t, _JPW)
        j = lax.rem(t, _JPW)
        pltpu.async_copy(
            g_hbm.at[wid_v.at[c, pl.ds(j * _CHUNK, _CHUNK)]],
            g_v.at[lax.rem(t, 4)],
            gsem)

    fire_gather(0)
    fire_gather(1)
    fire_gather(2)

    rowvecs = [_iota() + (16 * k) for k in range(_CHUNK // 16)]

    def step(t, carry):
        buf = lax.rem(t, 4)
        pltpu.make_async_copy(
            g_hbm.at[wid_v.at[0, pl.ds(0, _CHUNK)]], g_v.at[0], gsem).wait()

        @pl.when(t + 3 < _NSTEP)
        def _():
            fire_gather(t + 3)

        @pl.when(t >= 2)
        def _():
            pltpu.make_async_copy(
                a_v.at[0], out_hbm.at[0, :, pl.ds(b0, _CHUNK)], osem).wait()

        c = lax.div(t, _JPW)
        j = lax.rem(t, _JPW)

        buf2 = lax.rem(t, 2)
        for k in range(_CHUNK // 16):
            idxv = idx_v[c, pl.ds(j * _CHUNK + 16 * k, 16)]
            lane0 = (idxv & 3) * 32
            rows = rowvecs[k]
            for d0 in range(0, _D, 8):
                vals = [plsc.load_gather(g_v.at[buf], [rows, lane0 + (d0 + u)])
                        for u in range(8)]
                for u in range(8):
                    a_v[buf2, d0 + u, pl.ds(16 * k, 16)] = vals[u]

        pltpu.async_copy(
            a_v.at[buf2],
            out_hbm.at[c, :, pl.ds(b0 + j * _CHUNK, _CHUNK)],
            osem)
        return carry

    lax.fori_loop(0, _NSTEP, step, 0)
    pltpu.make_async_copy(
        a_v.at[0], out_hbm.at[0, :, pl.ds(b0, _CHUNK)], osem).wait()
    pltpu.make_async_copy(
        a_v.at[0], out_hbm.at[0, :, pl.ds(b0, _CHUNK)], osem).wait()


def kernel(input_idx, embedding_table):
    g = _format_table(embedding_table.T)
    out_t = _embed_gather(input_idx.T, g)
    return out_t.transpose(2, 0, 1)


# speedup vs baseline: 2.5213x; 2.1580x over previous
import functools

import jax
import jax.numpy as jnp
import numpy as np
from jax import lax
from jax.experimental import pallas as pl
from jax.experimental.pallas import tpu as pltpu
from jax.experimental.pallas import tpu_sc as plsc

_NC, _NS = 2, 16
_NW = _NC * _NS
_D = 32
_DPAD = 128
_V = 1000000
_NBLK = (_V + _DPAD - 1) // _DPAD
_GROWS = _NBLK * (_DPAD // 4)
_NB = 16384
_NCOL = 26
_BPW = _NB // _NW
_CHUNK = 128
_JPW = _BPW // _CHUNK
_NSTEP = _NCOL * _JPW

_mesh = plsc.VectorSubcoreMesh(core_axis_name="c", subcore_axis_name="s")


def _iota():
    return jnp.arange(16, dtype=jnp.int32)


@functools.partial(
    pl.kernel,
    mesh=_mesh,
    out_type=jax.ShapeDtypeStruct((_GROWS, _DPAD), jnp.float32),
    scratch_types=[
        pltpu.VMEM((2, _D, _DPAD), jnp.float32),
        pltpu.VMEM((2, _D, _DPAD), jnp.float32),
        pltpu.SemaphoreType.DMA,
        pltpu.SemaphoreType.DMA,
    ],
    compiler_params=pltpu.CompilerParams(
        use_tc_tiling_on_sc=True, disable_bounds_checks=True,
        needs_layout_passes=False),
)
def _format_table(tabt_hbm, g_hbm, s_v, t_v, isem, osem):
    wid = lax.axis_index("s") * _NC + lax.axis_index("c")
    nmine = lax.select(wid < _NBLK - (_NBLK // _NW) * _NW,
                       _NBLK // _NW + 1, _NBLK // _NW)

    def fire_in(m):
        blk = wid + m * _NW
        pltpu.async_copy(
            tabt_hbm.at[:, pl.ds(blk * _DPAD, _DPAD)],
            s_v.at[lax.rem(m, 2)],
            isem)

    @pl.when(nmine > 0)
    def _():
        fire_in(0)

    rowpat = _iota() >> 2
    colpat = _iota() & 3
    rowpats = [rowpat + (4 * g) for g in range(_DPAD // 16)]
    colpats = [colpat + (4 * d) for d in range(_D)]

    def block_step(m, carry):
        buf = lax.rem(m, 2)
        blk = wid + m * _NW
        pltpu.make_async_copy(
            tabt_hbm.at[:, pl.ds(0, _DPAD)], s_v.at[0], isem).wait()

        @pl.when(m + 1 < nmine)
        def _():
            fire_in(m + 1)

        @pl.when(m >= 2)
        def _():
            pltpu.make_async_copy(
                t_v.at[0], g_hbm.at[pl.ds(0, _D), :], osem).wait()

        for d in range(_D):
            vs = [s_v[buf, d, pl.ds(16 * g, 16)] for g in range(_DPAD // 16)]
            for g in range(_DPAD // 16):
                plsc.store_scatter(t_v.at[buf], [rowpats[g], colpats[d]], vs[g])

        pltpu.async_copy(
            t_v.at[buf],
            g_hbm.at[pl.ds(blk * _D, _D), :],
            osem)
        return carry

    lax.fori_loop(0, nmine, block_step, 0)
    @pl.when(nmine >= 1)
    def _():
        pltpu.make_async_copy(
            t_v.at[0], g_hbm.at[pl.ds(0, _D), :], osem).wait()

    @pl.when(nmine >= 2)
    def _():
        pltpu.make_async_copy(
            t_v.at[0], g_hbm.at[pl.ds(0, _D), :], osem).wait()


@functools.partial(
    pl.kernel,
    mesh=_mesh,
    out_type=jax.ShapeDtypeStruct((_NCOL, _D, _NB), jnp.float32),
    scratch_types=[
        pltpu.VMEM((_NCOL, _BPW), jnp.int32),
        pltpu.VMEM((_NCOL, _BPW), jnp.int32),
        pltpu.VMEM((4, _CHUNK, _DPAD), jnp.float32),
        pltpu.VMEM((2, _D, _CHUNK), jnp.float32),
        pltpu.SemaphoreType.DMA,
        pltpu.SemaphoreType.DMA,
    ],
    compiler_params=pltpu.CompilerParams(
        use_tc_tiling_on_sc=True, needs_layout_passes=False),
)
def _embed_gather(idxt_hbm, g_hbm, out_hbm, idx_v, wid_v, g_v, a_v, gsem, osem):
    wid = lax.axis_index("s") * _NC + lax.axis_index("c")
    b0 = wid * _BPW
    pltpu.sync_copy(idxt_hbm.at[:, pl.ds(b0, _BPW)], idx_v)

    def shift_row(c, carry):
        def shift_grp(k, carry2):
            v = idx_v[c, pl.ds(16 * k, 16)]
            wid_v[c, pl.ds(16 * k, 16)] = v >> 2
            return carry2
        lax.fori_loop(0, _BPW // 16, shift_grp, 0)
        return carry
    lax.fori_loop(0, _NCOL, shift_row, 0)

    def fire_gather(t):
        c = lax.div(t, _JPW)
        j = lax.rem(t, _JPW)
        pltpu.async_copy(
            g_hbm.at[wid_v.at[c, pl.ds(j * _CHUNK, _CHUNK)]],
            g_v.at[lax.rem(t, 4)],
            gsem)

    fire_gather(0)
    fire_gather(1)
    fire_gather(2)

    rowvecs = [_iota() + (16 * k) for k in range(_CHUNK // 16)]

    def step(t, carry):
        buf = lax.rem(t, 4)
        pltpu.make_async_copy(
            g_hbm.at[wid_v.at[0, pl.ds(0, _CHUNK)]], g_v.at[0], gsem).wait()

        @pl.when(t + 3 < _NSTEP)
        def _():
            fire_gather(t + 3)

        @pl.when(t >= 2)
        def _():
            pltpu.make_async_copy(
                a_v.at[0], out_hbm.at[0, :, pl.ds(b0, _CHUNK)], osem).wait()

        c = lax.div(t, _JPW)
        j = lax.rem(t, _JPW)

        buf2 = lax.rem(t, 2)
        for k in range(_CHUNK // 16):
            idxv = idx_v[c, pl.ds(j * _CHUNK + 16 * k, 16)]
            lane0 = idxv & 3
            rows = rowvecs[k]
            for d0 in range(0, _D, 8):
                vals = [plsc.load_gather(
                            g_v.at[buf], [rows, lane0 + (4 * (d0 + u))])
                        for u in range(8)]
                for u in range(8):
                    a_v[buf2, d0 + u, pl.ds(16 * k, 16)] = vals[u]

        pltpu.async_copy(
            a_v.at[buf2],
            out_hbm.at[c, :, pl.ds(b0 + j * _CHUNK, _CHUNK)],
            osem)
        return carry

    lax.fori_loop(0, _NSTEP, step, 0)
    pltpu.make_async_copy(
        a_v.at[0], out_hbm.at[0, :, pl.ds(b0, _CHUNK)], osem).wait()
    pltpu.make_async_copy(
        a_v.at[0], out_hbm.at[0, :, pl.ds(b0, _CHUNK)], osem).wait()


def kernel(input_idx, embedding_table):
    g = _format_table(embedding_table.T)
    out_t = _embed_gather(input_idx.T, g)
    return out_t.transpose(2, 0, 1)
